# baseline (speedup 1.0000x reference)
import jax
import jax.numpy as jnp
import numpy as np
from jax.experimental import pallas as pl
from jax.experimental.pallas import tpu as pltpu

N, C, Hh, L, P = 4, 256, 8, 4, 4
SHAPES_NP = np.array([[64, 64], [32, 32], [16, 16], [8, 8]], dtype=np.int64)


def _bilinear(img, x, y):
    B, H, W, D = img.shape
    x0 = jnp.floor(x); y0 = jnp.floor(y)
    x1 = x0 + 1.0; y1 = y0 + 1.0
    wx1 = x - x0; wx0 = 1.0 - wx1
    wy1 = y - y0; wy0 = 1.0 - wy1
    flat = img.reshape(B, H * W, D)

    def g(xi, yi):
        valid = (xi >= 0) & (xi <= W - 1) & (yi >= 0) & (yi <= H - 1)
        xc = jnp.clip(xi, 0, W - 1).astype(jnp.int32)
        yc = jnp.clip(yi, 0, H - 1).astype(jnp.int32)
        idx = yc * W + xc
        v = jnp.take_along_axis(flat, idx[..., None], axis=1)
        return v * valid[..., None].astype(v.dtype)

    return (g(x0, y0) * (wx0 * wy0)[..., None] + g(x1, y0) * (wx1 * wy0)[..., None]
            + g(x0, y1) * (wx0 * wy1)[..., None] + g(x1, y1) * (wx1 * wy1)[..., None])


def _matmul_bias_kernel(x_ref, w_ref, b_ref, o_ref):
    o_ref[...] = jnp.dot(x_ref[...], w_ref[...],
                         preferred_element_type=jnp.float32) + b_ref[...]


def _proj(x2d, W, b):
    R = x2d.shape[0]
    BLK = 680
    grid = (R // BLK,)
    return pl.pallas_call(
        _matmul_bias_kernel,
        grid=grid,
        in_specs=[
            pl.BlockSpec((BLK, C), lambda i: (i, 0)),
            pl.BlockSpec((C, C), lambda i: (0, 0)),
            pl.BlockSpec((1, C), lambda i: (0, 0)),
        ],
        out_specs=pl.BlockSpec((BLK, C), lambda i: (i, 0)),
        out_shape=jax.ShapeDtypeStruct((R, C), jnp.float32),
    )(x2d, W, b.reshape(1, C))


def kernel(query, reference_points, input_flatten, input_spatial_shapes,
           input_level_start_index, Wv, bv, Ws, bs, Wa, ba, Wo, bo):
    Nb, Lq, Cm = query.shape
    Dh = Cm // Hh
    value = input_flatten @ Wv + bv
    value = value.reshape(Nb, -1, Hh, Dh)
    off = (query @ Ws + bs).reshape(Nb, Lq, Hh, L, P, 2)
    aw = (query @ Wa + ba).reshape(Nb, Lq, Hh, L * P)
    aw = jax.nn.softmax(aw, axis=-1).reshape(Nb, Lq, Hh, L, P)
    norm = jnp.stack([input_spatial_shapes[:, 1], input_spatial_shapes[:, 0]], -1).astype(jnp.float32)
    loc = reference_points[:, :, None, :, None, :] + off / norm[None, None, None, :, None, :]
    sampled = []
    for l in range(L):
        H_, W_ = int(SHAPES_NP[l, 0]), int(SHAPES_NP[l, 1])
        v = jax.lax.dynamic_slice_in_dim(value, input_level_start_index[l], H_ * W_, axis=1)
        v = v.transpose(0, 2, 1, 3).reshape(Nb * Hh, H_, W_, Dh)
        ll = loc[:, :, :, l]
        ll = ll.transpose(0, 2, 1, 3, 4).reshape(Nb * Hh, Lq * P, 2)
        x = ll[..., 0] * W_ - 0.5
        y = ll[..., 1] * H_ - 0.5
        sampled.append(_bilinear(v, x, y).reshape(Nb * Hh, Lq, P, Dh))
    samp = jnp.stack(sampled, axis=2)
    w = aw.transpose(0, 2, 1, 3, 4).reshape(Nb * Hh, Lq, L, P, 1)
    out = (samp * w).sum(axis=(2, 3))
    out = out.reshape(Nb, Hh, Lq, Dh).transpose(0, 2, 1, 3).reshape(Nb, Lq, Cm)
    out = _proj(out.reshape(Nb * Lq, Cm), Wo, bo).reshape(Nb, Lq, Cm)
    return out


# speedup vs baseline: 64.5042x; 64.5042x over previous
import functools

import jax
import jax.numpy as jnp
import numpy as np
from jax import lax
from jax.experimental import pallas as pl
from jax.experimental.pallas import tpu as pltpu
from jax.experimental.pallas import tpu_sc as plsc

N, C, Hh, L, P = 4, 256, 8, 4, 4
Dh = C // Hh
SHAPES_NP = np.array([[64, 64], [32, 32], [16, 16], [8, 8]], dtype=np.int64)
LEN_IN = int((SHAPES_NP[:, 0] * SHAPES_NP[:, 1]).sum())
LSI_NP = np.concatenate([[0], np.cumsum(SHAPES_NP[:, 0] * SHAPES_NP[:, 1])[:-1]])
Lq = LEN_IN
NQ = N * Lq
NW = 32
QPT = NQ // NW
QB = 680
S = Hh * L * P

_s = np.arange(S)
_s_h = _s // (L * P)
_s_l = (_s % (L * P)) // P
_LANE_WF = SHAPES_NP[:, 1][_s_l].astype(np.float32)[None, :]
_LANE_HF = SHAPES_NP[:, 0][_s_l].astype(np.float32)[None, :]
_LANE_WI = SHAPES_NP[:, 1][_s_l].astype(np.int32)[None, :]
_LANE_HI = SHAPES_NP[:, 0][_s_l].astype(np.int32)[None, :]
_LANE_BASE = (LSI_NP[_s_l] + _s_h * LEN_IN).astype(np.int32)[None, :]
_SELX = np.zeros((8, S), np.float32)
_SELY = np.zeros((8, S), np.float32)
_SELX[2 * _s_l, _s] = 1.0
_SELY[2 * _s_l + 1, _s] = 1.0
_BONES = (np.arange(S)[:, None] // 16 == np.arange(S)[None, :] // 16).astype(np.float32)


def _prep_body(q_ref, rp_ref, wsx_ref, bsx_ref, wsy_ref, bsy_ref,
               wa_ref, ba_ref, bones_ref, selx_ref, sely_ref,
               lwf_ref, lhf_ref, lwi_ref, lhi_ref, lbase_ref,
               idx_ref, w_ref):
    n = pl.program_id(0)
    qb = q_ref[0]
    rp8 = rp_ref[0]
    f32 = jnp.float32
    offx = jnp.dot(qb, wsx_ref[...], preferred_element_type=f32) + bsx_ref[...]
    offy = jnp.dot(qb, wsy_ref[...], preferred_element_type=f32) + bsy_ref[...]
    a = jnp.dot(qb, wa_ref[...], preferred_element_type=f32) + ba_ref[...]
    m = jnp.max(a, axis=-1, keepdims=True)
    e = jnp.exp(a - m)
    hi = jax.lax.Precision.HIGHEST
    den = jnp.dot(e, bones_ref[...], preferred_element_type=f32, precision=hi)
    wa = e / den
    rpx = jnp.dot(rp8, selx_ref[...], preferred_element_type=f32, precision=hi)
    rpy = jnp.dot(rp8, sely_ref[...], preferred_element_type=f32, precision=hi)
    x = rpx * lwf_ref[...] + offx - 0.5
    y = rpy * lhf_ref[...] + offy - 0.5
    x0f = jnp.floor(x)
    y0f = jnp.floor(y)
    wx1 = x - x0f
    wx0 = 1.0 - wx1
    wy1 = y - y0f
    wy0 = 1.0 - wy1
    x0 = x0f.astype(jnp.int32)
    y0 = y0f.astype(jnp.int32)
    x1 = x0 + 1
    y1 = y0 + 1
    Wi = lwi_ref[...]
    Hi = lhi_ref[...]
    vx0 = ((x0 >= 0) & (x0 < Wi)).astype(f32)
    vx1 = ((x1 >= 0) & (x1 < Wi)).astype(f32)
    vy0 = ((y0 >= 0) & (y0 < Hi)).astype(f32)
    vy1 = ((y1 >= 0) & (y1 < Hi)).astype(f32)
    xc0 = jnp.clip(x0, 0, Wi - 1)
    xc1 = jnp.clip(x1, 0, Wi - 1)
    yc0 = jnp.clip(y0, 0, Hi - 1)
    yc1 = jnp.clip(y1, 0, Hi - 1)
    base = lbase_ref[...] + n * (Hh * LEN_IN)
    r0 = base + yc0 * Wi
    r1 = base + yc1 * Wi
    idx_ref[0] = jnp.concatenate(
        [r0 + xc0, r0 + xc1, r1 + xc0, r1 + xc1], axis=-1)
    wa0 = wa * wy0 * vy0
    wa1 = wa * wy1 * vy1
    w_ref[0] = jnp.concatenate(
        [wa0 * wx0 * vx0, wa0 * wx1 * vx1, wa1 * wx0 * vx0, wa1 * wx1 * vx1],
        axis=-1)


def _prep(query, rp8, Wsx, bsx, Wsy, bsy, Wa, ba):
    grid = (N, Lq // QB)
    full = lambda shp: pl.BlockSpec(shp, lambda n, j: tuple(0 for _ in shp))
    return pl.pallas_call(
        _prep_body,
        grid=grid,
        in_specs=[
            pl.BlockSpec((1, QB, C), lambda n, j: (n, j, 0)),
            pl.BlockSpec((1, QB, 8), lambda n, j: (n, j, 0)),
            full((C, S)), full((1, S)), full((C, S)), full((1, S)),
            full((C, S)), full((1, S)),
            full((S, S)), full((8, S)), full((8, S)),
            full((1, S)), full((1, S)), full((1, S)), full((1, S)),
            full((1, S)),
        ],
        out_specs=[
            pl.BlockSpec((1, QB, 4 * S), lambda n, j: (n, j, 0)),
            pl.BlockSpec((1, QB, 4 * S), lambda n, j: (n, j, 0)),
        ],
        out_shape=[
            jax.ShapeDtypeStruct((N, Lq, 4 * S), jnp.int32),
            jax.ShapeDtypeStruct((N, Lq, 4 * S), jnp.float32),
        ],
    )(query, rp8, Wsx, bsx, Wsy, bsy, Wa, ba,
      jnp.asarray(_BONES), jnp.asarray(_SELX), jnp.asarray(_SELY),
      jnp.asarray(_LANE_WF), jnp.asarray(_LANE_HF),
      jnp.asarray(_LANE_WI), jnp.asarray(_LANE_HI),
      jnp.asarray(_LANE_BASE))


def _value_body(x_ref, wv_ref, bv_ref, t_ref):
    t_ref[0, 0] = (jnp.dot(x_ref[0], wv_ref[0],
                           preferred_element_type=jnp.float32) + bv_ref[0])


def _value_table(input_flatten, Wv, bv):
    Wvh = Wv.reshape(C, Hh, Dh).transpose(1, 0, 2)
    bvh = bv.reshape(Hh, 1, Dh)
    return pl.pallas_call(
        _value_body,
        grid=(N, Hh),
        in_specs=[
            pl.BlockSpec((1, LEN_IN, C), lambda n, h: (n, 0, 0)),
            pl.BlockSpec((1, C, Dh), lambda n, h: (h, 0, 0)),
            pl.BlockSpec((1, 1, Dh), lambda n, h: (h, 0, 0)),
        ],
        out_specs=pl.BlockSpec((1, 1, LEN_IN, Dh), lambda n, h: (n, h, 0, 0)),
        out_shape=jax.ShapeDtypeStruct((N, Hh, LEN_IN, Dh), jnp.float32),
    )(input_flatten, Wvh, bvh)


def _matmul_bias_kernel(x_ref, w_ref, b_ref, o_ref):
    o_ref[...] = jnp.dot(x_ref[...], w_ref[...],
                         preferred_element_type=jnp.float32) + b_ref[...]


def _proj(x2d, W, b):
    R = x2d.shape[0]
    BLK = 680
    return pl.pallas_call(
        _matmul_bias_kernel,
        grid=(R // BLK,),
        in_specs=[
            pl.BlockSpec((BLK, C), lambda i: (i, 0)),
            pl.BlockSpec((C, C), lambda i: (0, 0)),
            pl.BlockSpec((1, C), lambda i: (0, 0)),
        ],
        out_specs=pl.BlockSpec((BLK, C), lambda i: (i, 0)),
        out_shape=jax.ShapeDtypeStruct((R, C), jnp.float32),
    )(x2d, W, b.reshape(1, C))


_SPLAT_DN = jax.lax.GatherDimensionNumbers(
    offset_dims=(), collapsed_slice_dims=(0,), start_index_map=(0,))


def _splat(vec16, lane):
    idx = jnp.full((16, 1), lane, jnp.int32)
    return jax.lax.gather(vec16, idx, _SPLAT_DN, (1,),
                          mode=jax.lax.GatherScatterMode.PROMISE_IN_BOUNDS)


def _sc_gather_reduce(idx2, w2, table):
    mesh = plsc.VectorSubcoreMesh(core_axis_name="c", subcore_axis_name="s")

    @functools.partial(
        pl.kernel,
        mesh=mesh,
        compiler_params=pltpu.CompilerParams(use_tc_tiling_on_sc=False),
        out_type=jax.ShapeDtypeStruct((NQ, C), jnp.float32),
        scratch_types=[
            pltpu.VMEM((4, S), jnp.int32),
            pltpu.VMEM((4, S), jnp.float32),
            pltpu.VMEM((4, S, Dh), jnp.float32),
            pltpu.VMEM((C,), jnp.float32),
            pltpu.SemaphoreType.DMA,
        ],
    )
    def sc(idx_hbm, w_hbm, table_hbm, out_hbm, idxb, wb, G, ob, sem):
        wid = lax.axis_index("s") * 2 + lax.axis_index("c")
        q0 = wid * QPT

        def per_q(i, carry):
            q = q0 + i
            pltpu.sync_copy(idx_hbm.at[pl.ds(q * 4, 4)], idxb)
            pltpu.sync_copy(w_hbm.at[pl.ds(q * 4, 4)], wb)
            cps = [pltpu.async_copy(table_hbm.at[idxb.at[c]], G.at[c], sem)
                   for c in range(4)]
            for cp in cps:
                cp.wait()

            def per_h(h, carry2):
                acc0 = jnp.zeros((16,), jnp.float32)
                acc1 = jnp.zeros((16,), jnp.float32)
                for c in range(4):
                    wv = wb[c, pl.ds(h * 16, 16)]
                    for lp in range(16):
                        ws = _splat(wv, lp)
                        e = h * 16 + lp
                        acc0 = acc0 + ws * G[c, e, pl.ds(0, 16)]
                        acc1 = acc1 + ws * G[c, e, pl.ds(16, 16)]
                ob[pl.ds(h * Dh, 16)] = acc0
                ob[pl.ds(h * Dh + 16, 16)] = acc1
                return carry2

            lax.fori_loop(0, Hh, per_h, 0)
            pltpu.sync_copy(ob, out_hbm.at[q])
            return carry

        lax.fori_loop(0, QPT, per_q, 0)

    return sc(idx2, w2, table)


def kernel(query, reference_points, input_flatten, input_spatial_shapes,
           input_level_start_index, Wv, bv, Ws, bs, Wa, ba, Wo, bo):
    rp8 = reference_points.reshape(N, Lq, L * 2)
    Ws3 = Ws.reshape(C, S, 2)
    bs2 = bs.reshape(S, 2)
    Wsx = Ws3[:, :, 0]
    Wsy = Ws3[:, :, 1]
    bsx = bs2[:, 0].reshape(1, S)
    bsy = bs2[:, 1].reshape(1, S)

    idx3, w3 = _prep(query, rp8, Wsx, bsx, Wsy, bsy, Wa, ba.reshape(1, S))
    idx2 = idx3.reshape(NQ * 4, S)
    w2 = w3.reshape(NQ * 4, S)
    table = _value_table(input_flatten, Wv, bv).reshape(N * Hh * LEN_IN, Dh)
    hout = _sc_gather_reduce(idx2, w2, table)
    out = _proj(hout, Wo, bo)
    return out.reshape(N, Lq, C)


# speedup vs baseline: 137.7777x; 2.1359x over previous
import functools

import jax
import jax.numpy as jnp
import numpy as np
from jax import lax
from jax.experimental import pallas as pl
from jax.experimental.pallas import tpu as pltpu
from jax.experimental.pallas import tpu_sc as plsc

N, C, Hh, L, P = 4, 256, 8, 4, 4
Dh = C // Hh
SHAPES_NP = np.array([[64, 64], [32, 32], [16, 16], [8, 8]], dtype=np.int64)
LEN_IN = int((SHAPES_NP[:, 0] * SHAPES_NP[:, 1]).sum())
LSI_NP = np.concatenate([[0], np.cumsum(SHAPES_NP[:, 0] * SHAPES_NP[:, 1])[:-1]])
Lq = LEN_IN
NQ = N * Lq
NW = 32
QPT = NQ // NW
QB = 680
S = Hh * L * P

_s = np.arange(S)
_s_h = _s // (L * P)
_s_l = (_s % (L * P)) // P
_LANE_WF = SHAPES_NP[:, 1][_s_l].astype(np.float32)[None, :]
_LANE_HF = SHAPES_NP[:, 0][_s_l].astype(np.float32)[None, :]
_LANE_WI = SHAPES_NP[:, 1][_s_l].astype(np.int32)[None, :]
_LANE_HI = SHAPES_NP[:, 0][_s_l].astype(np.int32)[None, :]
_LANE_BASE = (LSI_NP[_s_l] + _s_h * LEN_IN).astype(np.int32)[None, :]
_SELX = np.zeros((8, S), np.float32)
_SELY = np.zeros((8, S), np.float32)
_SELX[2 * _s_l, _s] = 1.0
_SELY[2 * _s_l + 1, _s] = 1.0
_BONES = (np.arange(S)[:, None] // 16 == np.arange(S)[None, :] // 16).astype(np.float32)


def _prep_body(q_ref, rp_ref, wsx_ref, bsx_ref, wsy_ref, bsy_ref,
               wa_ref, ba_ref, bones_ref, selx_ref, sely_ref,
               lwf_ref, lhf_ref, lwi_ref, lhi_ref, lbase_ref,
               iw_ref):
    n = pl.program_id(0)
    qb = q_ref[0]
    rp8 = rp_ref[0]
    f32 = jnp.float32
    offx = jnp.dot(qb, wsx_ref[...], preferred_element_type=f32) + bsx_ref[...]
    offy = jnp.dot(qb, wsy_ref[...], preferred_element_type=f32) + bsy_ref[...]
    a = jnp.dot(qb, wa_ref[...], preferred_element_type=f32) + ba_ref[...]
    m = jnp.max(a, axis=-1, keepdims=True)
    e = jnp.exp(a - m)
    hi = jax.lax.Precision.HIGHEST
    den = jnp.dot(e, bones_ref[...], preferred_element_type=f32, precision=hi)
    wa = e / den
    rpx = jnp.dot(rp8, selx_ref[...], preferred_element_type=f32, precision=hi)
    rpy = jnp.dot(rp8, sely_ref[...], preferred_element_type=f32, precision=hi)
    x = rpx * lwf_ref[...] + offx - 0.5
    y = rpy * lhf_ref[...] + offy - 0.5
    x0f = jnp.floor(x)
    y0f = jnp.floor(y)
    wx1 = x - x0f
    wx0 = 1.0 - wx1
    wy1 = y - y0f
    wy0 = 1.0 - wy1
    x0 = x0f.astype(jnp.int32)
    y0 = y0f.astype(jnp.int32)
    x1 = x0 + 1
    y1 = y0 + 1
    Wi = lwi_ref[...]
    Hi = lhi_ref[...]
    vx0 = ((x0 >= 0) & (x0 < Wi)).astype(f32)
    vx1 = ((x1 >= 0) & (x1 < Wi)).astype(f32)
    vy0 = ((y0 >= 0) & (y0 < Hi)).astype(f32)
    vy1 = ((y1 >= 0) & (y1 < Hi)).astype(f32)
    xc0 = jnp.clip(x0, 0, Wi - 1)
    xc1 = jnp.clip(x1, 0, Wi - 1)
    yc0 = jnp.clip(y0, 0, Hi - 1)
    yc1 = jnp.clip(y1, 0, Hi - 1)
    base = lbase_ref[...] + n * (Hh * LEN_IN)
    r0 = base + yc0 * Wi
    r1 = base + yc1 * Wi
    wa0 = wa * wy0 * vy0
    wa1 = wa * wy1 * vy1
    bc = lambda v: (v * jnp.float32(2.0 ** 23)).astype(jnp.int32)
    iw_ref[0, :, 0, :] = r0 + xc0
    iw_ref[0, :, 1, :] = r0 + xc1
    iw_ref[0, :, 2, :] = r1 + xc0
    iw_ref[0, :, 3, :] = r1 + xc1
    iw_ref[0, :, 4, :] = bc(wa0 * wx0 * vx0)
    iw_ref[0, :, 5, :] = bc(wa0 * wx1 * vx1)
    iw_ref[0, :, 6, :] = bc(wa1 * wx0 * vx0)
    iw_ref[0, :, 7, :] = bc(wa1 * wx1 * vx1)


def _prep(query, rp8, Wsx, bsx, Wsy, bsy, Wa, ba):
    grid = (N, Lq // QB)
    full = lambda shp: pl.BlockSpec(shp, lambda n, j: tuple(0 for _ in shp))
    return pl.pallas_call(
        _prep_body,
        grid=grid,
        in_specs=[
            pl.BlockSpec((1, QB, C), lambda n, j: (n, j, 0)),
            pl.BlockSpec((1, QB, 8), lambda n, j: (n, j, 0)),
            full((C, S)), full((1, S)), full((C, S)), full((1, S)),
            full((C, S)), full((1, S)),
            full((S, S)), full((8, S)), full((8, S)),
            full((1, S)), full((1, S)), full((1, S)), full((1, S)),
            full((1, S)),
        ],
        out_specs=pl.BlockSpec((1, QB, 8, S), lambda n, j: (n, j, 0, 0)),
        out_shape=jax.ShapeDtypeStruct((N, Lq, 8, S), jnp.int32),
    )(query, rp8, Wsx, bsx, Wsy, bsy, Wa, ba,
      jnp.asarray(_BONES), jnp.asarray(_SELX), jnp.asarray(_SELY),
      jnp.asarray(_LANE_WF), jnp.asarray(_LANE_HF),
      jnp.asarray(_LANE_WI), jnp.asarray(_LANE_HI),
      jnp.asarray(_LANE_BASE))


def _value_body(x_ref, wv_ref, bv_ref, t_ref):
    t_ref[0, 0] = (jnp.dot(x_ref[0], wv_ref[0],
                           preferred_element_type=jnp.float32) + bv_ref[0])


def _value_table(input_flatten, Wv, bv):
    Wvh = Wv.reshape(C, Hh, Dh).transpose(1, 0, 2)
    bvh = bv.reshape(Hh, 1, Dh)
    return pl.pallas_call(
        _value_body,
        grid=(N, Hh),
        in_specs=[
            pl.BlockSpec((1, LEN_IN, C), lambda n, h: (n, 0, 0)),
            pl.BlockSpec((1, C, Dh), lambda n, h: (h, 0, 0)),
            pl.BlockSpec((1, 1, Dh), lambda n, h: (h, 0, 0)),
        ],
        out_specs=pl.BlockSpec((1, 1, LEN_IN, Dh), lambda n, h: (n, h, 0, 0)),
        out_shape=jax.ShapeDtypeStruct((N, Hh, LEN_IN, Dh), jnp.float32),
    )(input_flatten, Wvh, bvh)


def _matmul_bias_kernel(x_ref, w_ref, b_ref, o_ref):
    o_ref[...] = jnp.dot(x_ref[...], w_ref[...],
                         preferred_element_type=jnp.float32) + b_ref[...]


def _proj(x2d, W, b):
    R = x2d.shape[0]
    BLK = 680
    return pl.pallas_call(
        _matmul_bias_kernel,
        grid=(R // BLK,),
        in_specs=[
            pl.BlockSpec((BLK, C), lambda i: (i, 0)),
            pl.BlockSpec((C, C), lambda i: (0, 0)),
            pl.BlockSpec((1, C), lambda i: (0, 0)),
        ],
        out_specs=pl.BlockSpec((BLK, C), lambda i: (i, 0)),
        out_shape=jax.ShapeDtypeStruct((R, C), jnp.float32),
    )(x2d, W, b.reshape(1, C))


_SPLAT_DN = jax.lax.GatherDimensionNumbers(
    offset_dims=(), collapsed_slice_dims=(0,), start_index_map=(0,))


def _splat(vec16, lane):
    idx = jnp.full((16, 1), lane, jnp.int32)
    return jax.lax.gather(vec16, idx, _SPLAT_DN, (1,),
                          mode=jax.lax.GatherScatterMode.PROMISE_IN_BOUNDS)


NSLOT = 4
NITER = QPT // NSLOT


def _sc_gather_reduce(idxw, table):
    mesh = plsc.VectorSubcoreMesh(core_axis_name="c", subcore_axis_name="s")

    @functools.partial(
        pl.kernel,
        mesh=mesh,
        compiler_params=pltpu.CompilerParams(use_tc_tiling_on_sc=False),
        out_type=jax.ShapeDtypeStruct((NQ, C), jnp.float32),
        scratch_types=(
            [pltpu.VMEM((NSLOT, 8, S), jnp.int32),
             pltpu.VMEM((NSLOT, 4, S, Dh), jnp.float32),
             pltpu.VMEM((NSLOT, C), jnp.float32)]
            + [pltpu.SemaphoreType.DMA] * (3 * NSLOT)
        ),
    )
    def sc(idxw_hbm, table_hbm, out_hbm, iw, G, ob, *sems):
        iwsem = sems[0:NSLOT]
        gsem = sems[NSLOT:2 * NSLOT]
        osem = sems[2 * NSLOT:3 * NSLOT]
        wid = lax.axis_index("s") * 2 + lax.axis_index("c")
        q0 = wid * QPT

        def fire_gathers(s):
            for c in range(4):
                pltpu.async_copy(table_hbm.at[iw.at[s, c]], G.at[s, c],
                                 gsem[s])

        def wait_gathers(s):
            for c in range(4):
                pltpu.make_async_copy(table_hbm.at[pl.ds(0, S)], G.at[s, c],
                                      gsem[s]).wait()

        def compute(s):
            def per_h(h, carry):
                acc0 = jnp.zeros((16,), jnp.float32)
                acc1 = jnp.zeros((16,), jnp.float32)
                for c in range(4):
                    wv = iw[s, 4 + c, pl.ds(h * 16, 16)].astype(jnp.float32)
                    for lp in range(16):
                        ws = _splat(wv, lp)
                        e = h * 16 + lp
                        acc0 = acc0 + ws * G[s, c, e, pl.ds(0, 16)]
                        acc1 = acc1 + ws * G[s, c, e, pl.ds(16, 16)]
                sc_w = jnp.float32(2.0 ** -23)
                ob[s, pl.ds(h * Dh, 16)] = acc0 * sc_w
                ob[s, pl.ds(h * Dh + 16, 16)] = acc1 * sc_w
                return carry

            lax.fori_loop(0, Hh, per_h, 0)

        pltpu.sync_copy(idxw_hbm.at[pl.ds(q0 * 8, 8)], iw.at[0])
        fire_gathers(0)
        pltpu.async_copy(idxw_hbm.at[pl.ds((q0 + 1) * 8, 8)], iw.at[1],
                         iwsem[1])
        pltpu.async_copy(idxw_hbm.at[pl.ds((q0 + 2) * 8, 8)], iw.at[2],
                         iwsem[2])

        def body(i, carry):
            for b in range(NSLOT):
                q = q0 + NSLOT * i + b
                s1 = (b + 1) % NSLOT
                s3 = (b + 3) % NSLOT

                def arrive_and_fire():
                    pltpu.make_async_copy(idxw_hbm.at[pl.ds(0, 8)],
                                          iw.at[s1], iwsem[s1]).wait()
                    fire_gathers(s1)

                if b < NSLOT - 1:
                    arrive_and_fire()
                else:
                    pl.when(i < NITER - 1)(arrive_and_fire)

                wait_gathers(b)

                def prefetch():
                    pltpu.async_copy(idxw_hbm.at[pl.ds((q + 3) * 8, 8)],
                                     iw.at[s3], iwsem[s3])

                if b == 0:
                    prefetch()
                else:
                    pl.when(i < NITER - 1)(prefetch)

                @pl.when(i >= 1)
                def _drain_out():
                    pltpu.make_async_copy(out_hbm.at[0], ob.at[b],
                                          osem[b]).wait()

                compute(b)
                pltpu.async_copy(ob.at[b], out_hbm.at[q], osem[b])
            return carry

        lax.fori_loop(0, NITER, body, 0)
        for b in range(NSLOT):
            pltpu.make_async_copy(out_hbm.at[0], ob.at[b], osem[b]).wait()

    return sc(idxw, table)


def kernel(query, reference_points, input_flatten, input_spatial_shapes,
           input_level_start_index, Wv, bv, Ws, bs, Wa, ba, Wo, bo):
    rp8 = reference_points.reshape(N, Lq, L * 2)
    Ws3 = Ws.reshape(C, S, 2)
    bs2 = bs.reshape(S, 2)
    Wsx = Ws3[:, :, 0]
    Wsy = Ws3[:, :, 1]
    bsx = bs2[:, 0].reshape(1, S)
    bsy = bs2[:, 1].reshape(1, S)

    iw4 = _prep(query, rp8, Wsx, bsx, Wsy, bsy, Wa, ba.reshape(1, S))
    idxw = iw4.reshape(NQ * 8, S)
    table = _value_table(input_flatten, Wv, bv).reshape(N * Hh * LEN_IN, Dh)
    hout = _sc_gather_reduce(idxw, table)
    out = _proj(hout, Wo, bo)
    return out.reshape(N, Lq, C)


# speedup vs baseline: 150.7055x; 1.0938x over previous
import functools

import jax
import jax.numpy as jnp
import numpy as np
from jax import lax
from jax.experimental import pallas as pl
from jax.experimental.pallas import tpu as pltpu
from jax.experimental.pallas import tpu_sc as plsc

N, C, Hh, L, P = 4, 256, 8, 4, 4
Dh = C // Hh
SHAPES_NP = np.array([[64, 64], [32, 32], [16, 16], [8, 8]], dtype=np.int64)
LEN_IN = int((SHAPES_NP[:, 0] * SHAPES_NP[:, 1]).sum())
LSI_NP = np.concatenate([[0], np.cumsum(SHAPES_NP[:, 0] * SHAPES_NP[:, 1])[:-1]])
Lq = LEN_IN
NQ = N * Lq
NW = 32
QPT = NQ // NW
QB = 680
S = Hh * L * P

_s = np.arange(S)
_s_h = _s // (L * P)
_s_l = (_s % (L * P)) // P
_LANE_WF = SHAPES_NP[:, 1][_s_l].astype(np.float32)[None, :]
_LANE_HF = SHAPES_NP[:, 0][_s_l].astype(np.float32)[None, :]
_LANE_WI = SHAPES_NP[:, 1][_s_l].astype(np.int32)[None, :]
_LANE_HI = SHAPES_NP[:, 0][_s_l].astype(np.int32)[None, :]
_LANE_BASE = (LSI_NP[_s_l] + _s_h * LEN_IN).astype(np.int32)[None, :]
_SELX = np.zeros((8, S), np.float32)
_SELY = np.zeros((8, S), np.float32)
_SELX[2 * _s_l, _s] = 1.0
_SELY[2 * _s_l + 1, _s] = 1.0
_BONES = (np.arange(S)[:, None] // 16 == np.arange(S)[None, :] // 16).astype(np.float32)


def _prep_body(q_ref, rp_ref, wsx_ref, bsx_ref, wsy_ref, bsy_ref,
               wa_ref, ba_ref, bones_ref, selx_ref, sely_ref,
               lwf_ref, lhf_ref, lwi_ref, lhi_ref, lbase_ref,
               iw_ref):
    n = pl.program_id(0)
    qb = q_ref[0]
    rp8 = rp_ref[0]
    f32 = jnp.float32
    offx = jnp.dot(qb, wsx_ref[...], preferred_element_type=f32) + bsx_ref[...]
    offy = jnp.dot(qb, wsy_ref[...], preferred_element_type=f32) + bsy_ref[...]
    a = jnp.dot(qb, wa_ref[...], preferred_element_type=f32) + ba_ref[...]
    m = jnp.max(a, axis=-1, keepdims=True)
    e = jnp.exp(a - m)
    hi = jax.lax.Precision.HIGHEST
    den = jnp.dot(e, bones_ref[...], preferred_element_type=f32, precision=hi)
    wa = e / den
    rpx = jnp.dot(rp8, selx_ref[...], preferred_element_type=f32, precision=hi)
    rpy = jnp.dot(rp8, sely_ref[...], preferred_element_type=f32, precision=hi)
    x = rpx * lwf_ref[...] + offx - 0.5
    y = rpy * lhf_ref[...] + offy - 0.5
    x0f = jnp.floor(x)
    y0f = jnp.floor(y)
    wx1 = x - x0f
    wx0 = 1.0 - wx1
    wy1 = y - y0f
    wy0 = 1.0 - wy1
    x0 = x0f.astype(jnp.int32)
    y0 = y0f.astype(jnp.int32)
    x1 = x0 + 1
    y1 = y0 + 1
    Wi = lwi_ref[...]
    Hi = lhi_ref[...]
    vx0 = ((x0 >= 0) & (x0 < Wi)).astype(f32)
    vx1 = ((x1 >= 0) & (x1 < Wi)).astype(f32)
    vy0 = ((y0 >= 0) & (y0 < Hi)).astype(f32)
    vy1 = ((y1 >= 0) & (y1 < Hi)).astype(f32)
    xc0 = jnp.clip(x0, 0, Wi - 1)
    xc1 = jnp.clip(x1, 0, Wi - 1)
    yc0 = jnp.clip(y0, 0, Hi - 1)
    yc1 = jnp.clip(y1, 0, Hi - 1)
    base = lbase_ref[...] + n * (Hh * LEN_IN)
    r0 = base + yc0 * Wi
    r1 = base + yc1 * Wi
    wa0 = wa * wy0 * vy0
    wa1 = wa * wy1 * vy1
    bc = lambda v: (v * jnp.float32(2.0 ** 23)).astype(jnp.int32)
    iw_ref[0, :, 0, :] = r0 + xc0
    iw_ref[0, :, 1, :] = r0 + xc1
    iw_ref[0, :, 2, :] = r1 + xc0
    iw_ref[0, :, 3, :] = r1 + xc1
    iw_ref[0, :, 4, :] = bc(wa0 * wx0 * vx0)
    iw_ref[0, :, 5, :] = bc(wa0 * wx1 * vx1)
    iw_ref[0, :, 6, :] = bc(wa1 * wx0 * vx0)
    iw_ref[0, :, 7, :] = bc(wa1 * wx1 * vx1)


def _prep(query, rp8, Wsx, bsx, Wsy, bsy, Wa, ba):
    grid = (N, Lq // QB)
    full = lambda shp: pl.BlockSpec(shp, lambda n, j: tuple(0 for _ in shp))
    return pl.pallas_call(
        _prep_body,
        grid=grid,
        in_specs=[
            pl.BlockSpec((1, QB, C), lambda n, j: (n, j, 0)),
            pl.BlockSpec((1, QB, 8), lambda n, j: (n, j, 0)),
            full((C, S)), full((1, S)), full((C, S)), full((1, S)),
            full((C, S)), full((1, S)),
            full((S, S)), full((8, S)), full((8, S)),
            full((1, S)), full((1, S)), full((1, S)), full((1, S)),
            full((1, S)),
        ],
        out_specs=pl.BlockSpec((1, QB, 8, S), lambda n, j: (n, j, 0, 0)),
        out_shape=jax.ShapeDtypeStruct((N, Lq, 8, S), jnp.int32),
    )(query, rp8, Wsx, bsx, Wsy, bsy, Wa, ba,
      jnp.asarray(_BONES), jnp.asarray(_SELX), jnp.asarray(_SELY),
      jnp.asarray(_LANE_WF), jnp.asarray(_LANE_HF),
      jnp.asarray(_LANE_WI), jnp.asarray(_LANE_HI),
      jnp.asarray(_LANE_BASE))


_BF_PERM = np.ravel(np.column_stack([np.arange(16), np.arange(16) + 16]))


def _value_body(x_ref, wv_ref, bv_ref, t_ref):
    mm = (jnp.dot(x_ref[0], wv_ref[0],
                  preferred_element_type=jnp.float32) + bv_ref[0])
    t_ref[0, 0] = mm.astype(jnp.bfloat16)


def _value_table(input_flatten, Wv, bv):
    Wvh = Wv.reshape(C, Hh, Dh).transpose(1, 0, 2)[:, :, _BF_PERM]
    bvh = bv.reshape(Hh, 1, Dh)[:, :, _BF_PERM]
    return pl.pallas_call(
        _value_body,
        grid=(N, Hh),
        in_specs=[
            pl.BlockSpec((1, LEN_IN, C), lambda n, h: (n, 0, 0)),
            pl.BlockSpec((1, C, Dh), lambda n, h: (h, 0, 0)),
            pl.BlockSpec((1, 1, Dh), lambda n, h: (h, 0, 0)),
        ],
        out_specs=pl.BlockSpec((1, 1, LEN_IN, Dh), lambda n, h: (n, h, 0, 0)),
        out_shape=jax.ShapeDtypeStruct((N, Hh, LEN_IN, Dh), jnp.bfloat16),
    )(input_flatten, Wvh, bvh)


def _matmul_bias_kernel(x_ref, w_ref, b_ref, o_ref):
    o_ref[...] = jnp.dot(x_ref[...], w_ref[...],
                         preferred_element_type=jnp.float32) + b_ref[...]


def _proj(x2d, W, b):
    R = x2d.shape[0]
    BLK = 680
    return pl.pallas_call(
        _matmul_bias_kernel,
        grid=(R // BLK,),
        in_specs=[
            pl.BlockSpec((BLK, C), lambda i: (i, 0)),
            pl.BlockSpec((C, C), lambda i: (0, 0)),
            pl.BlockSpec((1, C), lambda i: (0, 0)),
        ],
        out_specs=pl.BlockSpec((BLK, C), lambda i: (i, 0)),
        out_shape=jax.ShapeDtypeStruct((R, C), jnp.float32),
    )(x2d, W, b.reshape(1, C))


_SPLAT_DN = jax.lax.GatherDimensionNumbers(
    offset_dims=(), collapsed_slice_dims=(0,), start_index_map=(0,))


def _splat(vec16, lane):
    idx = jnp.full((16, 1), lane, jnp.int32)
    return jax.lax.gather(vec16, idx, _SPLAT_DN, (1,),
                          mode=jax.lax.GatherScatterMode.PROMISE_IN_BOUNDS)


NSLOT = 4
NITER = QPT // NSLOT


def _sc_gather_reduce(idxw, table):
    mesh = plsc.VectorSubcoreMesh(core_axis_name="c", subcore_axis_name="s")

    @functools.partial(
        pl.kernel,
        mesh=mesh,
        compiler_params=pltpu.CompilerParams(use_tc_tiling_on_sc=False,
                                             needs_layout_passes=False),
        out_type=jax.ShapeDtypeStruct((NQ, C), jnp.float32),
        scratch_types=(
            [pltpu.VMEM((NSLOT, 8, S), jnp.int32),
             pltpu.VMEM((NSLOT, 4, S, Dh), jnp.bfloat16),
             pltpu.VMEM((NSLOT, C), jnp.float32)]
            + [pltpu.SemaphoreType.DMA] * (3 * NSLOT)
        ),
    )
    def sc(idxw_hbm, table_hbm, out_hbm, iw, G, ob, *sems):
        iwsem = sems[0:NSLOT]
        gsem = sems[NSLOT:2 * NSLOT]
        osem = sems[2 * NSLOT:3 * NSLOT]
        wid = lax.axis_index("s") * 2 + lax.axis_index("c")
        q0 = wid * QPT

        def fire_gathers(s):
            for c in range(4):
                pltpu.async_copy(table_hbm.at[iw.at[s, c]], G.at[s, c],
                                 gsem[s])

        def wait_gathers(s):
            for c in range(4):
                pltpu.make_async_copy(table_hbm.at[pl.ds(0, S)], G.at[s, c],
                                      gsem[s]).wait()

        def compute(s):
            def per_h(h, carry):
                acc0 = jnp.zeros((16,), jnp.float32)
                acc1 = jnp.zeros((16,), jnp.float32)
                for c in range(4):
                    wv = iw[s, 4 + c, pl.ds(h * 16, 16)].astype(jnp.float32)
                    for lp in range(16):
                        ws = _splat(wv, lp)
                        e = h * 16 + lp
                        row = G[s, c, e, :]
                        lo, hi = plsc.unpack(
                            row, format=plsc.PackFormat.INTERLEAVED)
                        acc0 = acc0 + ws * lo
                        acc1 = acc1 + ws * hi
                sc_w = jnp.float32(2.0 ** -23)
                ob[s, pl.ds(h * Dh, 16)] = acc0 * sc_w
                ob[s, pl.ds(h * Dh + 16, 16)] = acc1 * sc_w
                return carry

            lax.fori_loop(0, Hh, per_h, 0)

        pltpu.sync_copy(idxw_hbm.at[pl.ds(q0 * 8, 8)], iw.at[0])
        fire_gathers(0)
        pltpu.async_copy(idxw_hbm.at[pl.ds((q0 + 1) * 8, 8)], iw.at[1],
                         iwsem[1])
        pltpu.async_copy(idxw_hbm.at[pl.ds((q0 + 2) * 8, 8)], iw.at[2],
                         iwsem[2])

        def body(i, carry):
            for b in range(NSLOT):
                q = q0 + NSLOT * i + b
                s1 = (b + 1) % NSLOT
                s3 = (b + 3) % NSLOT

                def arrive_and_fire():
                    pltpu.make_async_copy(idxw_hbm.at[pl.ds(0, 8)],
                                          iw.at[s1], iwsem[s1]).wait()
                    fire_gathers(s1)

                if b < NSLOT - 1:
                    arrive_and_fire()
                else:
                    pl.when(i < NITER - 1)(arrive_and_fire)

                wait_gathers(b)

                def prefetch():
                    pltpu.async_copy(idxw_hbm.at[pl.ds((q + 3) * 8, 8)],
                                     iw.at[s3], iwsem[s3])

                if b == 0:
                    prefetch()
                else:
                    pl.when(i < NITER - 1)(prefetch)

                @pl.when(i >= 1)
                def _drain_out():
                    pltpu.make_async_copy(out_hbm.at[0], ob.at[b],
                                          osem[b]).wait()

                compute(b)
                pltpu.async_copy(ob.at[b], out_hbm.at[q], osem[b])
            return carry

        lax.fori_loop(0, NITER, body, 0)
        for b in range(NSLOT):
            pltpu.make_async_copy(out_hbm.at[0], ob.at[b], osem[b]).wait()

    return sc(idxw, table)


def kernel(query, reference_points, input_flatten, input_spatial_shapes,
           input_level_start_index, Wv, bv, Ws, bs, Wa, ba, Wo, bo):
    rp8 = reference_points.reshape(N, Lq, L * 2)
    Ws3 = Ws.reshape(C, S, 2)
    bs2 = bs.reshape(S, 2)
    Wsx = Ws3[:, :, 0]
    Wsy = Ws3[:, :, 1]
    bsx = bs2[:, 0].reshape(1, S)
    bsy = bs2[:, 1].reshape(1, S)

    iw4 = _prep(query, rp8, Wsx, bsx, Wsy, bsy, Wa, ba.reshape(1, S))
    idxw = iw4.reshape(NQ * 8, S)
    table = _value_table(input_flatten, Wv, bv).reshape(N * Hh * LEN_IN, Dh)
    hout = _sc_gather_reduce(idxw, table)
    out = _proj(hout, Wo, bo)
    return out.reshape(N, Lq, C)
